# Initial kernel scaffold; baseline (speedup 1.0000x reference)
import jax
import jax.numpy as jnp
from jax.experimental import pallas as pl


def kernel(des, tweet, num_prop, cat_prop, edge_index, edge_type, W_des, b_des, W_num, b_num, W_cat, b_cat, W_tweet, b_tweet, W_in, b_in, W_root, W_rel, b_rgcn, W_out1, b_out1, W_out2, b_out2):
    raise NotImplementedError("write your pallas kernel here")


# speedup vs baseline: 1.7275x; 1.7275x over previous
import functools

import jax
import jax.numpy as jnp
from jax import lax
from jax.experimental import pallas as pl
from jax.experimental.pallas import tpu as pltpu
from jax.experimental.pallas import tpu_sc as plsc

N = 20000
E = 320000
D = 128

NC, NS, NW = 2, 16, 32
R = 10000
ACCR = 10240
ZSPAN = 640
WSPAN = 624
WTAIL = R - NS * WSPAN
CHUNK = 128
EB = 2048
E_PAD = 327680
EPP = E_PAD // NS

_SDS = jax.ShapeDtypeStruct


def _lk(x):
    return jnp.where(x >= 0, x, 0.01 * x)


BT = 1000


def _prep_body(p_ref, w1_ref, b1_ref, win_ref, bin_ref, o_ref):
    x = p_ref[...]
    h = _lk(jnp.dot(x, w1_ref[0], preferred_element_type=jnp.float32)
            + b1_ref[0, 0])
    o_ref[...] = _lk(jnp.dot(h, win_ref[...],
                             preferred_element_type=jnp.float32)
                     + bin_ref[0])


def _prep(P, W1, b1, W_in, b_in):
    nb = N // BT
    return pl.pallas_call(
        _prep_body,
        out_shape=_SDS((N, D), jnp.float32),
        grid=(nb,),
        in_specs=[
            pl.BlockSpec((BT, 128), lambda i: (i, 0)),
            pl.BlockSpec((1, 128, 128),
                         lambda i: (lax.div(i, nb // 2), 0, 0)),
            pl.BlockSpec((1, 1, 128),
                         lambda i: (lax.div(i, nb // 2), 0, 0)),
            pl.BlockSpec((128, 128), lambda i: (0, 0)),
            pl.BlockSpec((1, 128), lambda i: (0, 0)),
        ],
        out_specs=pl.BlockSpec((BT, D), lambda i: (i, 0)),
    )(P, W1, b1, W_in, b_in)


def _combine_body(head, xa_ref, a0_ref, a1_ref, c0_ref, c1_ref,
                  wr_ref, w0_ref, w1_ref, b_ref,
                  wo1_ref, bo1_ref, wo2_ref, bo2_ref, o_ref):
    x = xa_ref[...]
    m0 = a0_ref[...] / jnp.maximum(c0_ref[:, 0:1], 1.0)
    m1 = a1_ref[...] / jnp.maximum(c1_ref[:, 0:1], 1.0)
    h = (jnp.dot(x, wr_ref[...], preferred_element_type=jnp.float32)
         + jnp.dot(m0, w0_ref[...], preferred_element_type=jnp.float32)
         + jnp.dot(m1, w1_ref[...], preferred_element_type=jnp.float32)
         + b_ref[0])
    if head:
        g = _lk(jnp.dot(h, wo1_ref[...], preferred_element_type=jnp.float32)
                + bo1_ref[0])
        o_ref[...] = (jnp.dot(g, wo2_ref[...],
                              preferred_element_type=jnp.float32)
                      + bo2_ref[0])
    else:
        o_ref[...] = h


def _combine(head, xa, acc, cnt, W_root, W0, W1, b, Wo1, bo1, Wo2, bo2):
    nb = N // BT
    out_w = 2 if head else D
    blk = pl.BlockSpec((BT, D), lambda i: (i, 0))
    blk2 = pl.BlockSpec((BT, D), lambda i, nb=nb: (i + nb, 0))
    wmat = pl.BlockSpec((128, 128), lambda i: (0, 0))
    wrow = pl.BlockSpec((1, 128), lambda i: (0, 0))
    return pl.pallas_call(
        functools.partial(_combine_body, head),
        out_shape=_SDS((N, out_w), jnp.float32),
        grid=(nb,),
        in_specs=[
            blk, blk, blk2, blk, blk2,
            wmat, wmat, wmat, wrow,
            wmat, wrow,
            pl.BlockSpec((128, 2), lambda i: (0, 0)),
            pl.BlockSpec((1, 2), lambda i: (0, 0)),
        ],
        out_specs=pl.BlockSpec((BT, out_w), lambda i: (i, 0)),
    )(xa, acc, acc, cnt, cnt, W_root, W0, W1, b, Wo1, bo1, Wo2, bo2)


def _compute_loc(edv, etv, locb, j, lo, dummy):
    for g in range(CHUNK // 16):
        off = pl.ds(j * CHUNK + g * 16, 16)
        ai = etv[off] * N + edv[off]
        inr = (ai >= lo) & (ai < lo + R)
        locb[pl.ds(g * 16, 16)] = jnp.where(inr, ai - lo, dummy)


def _sc_agg_body(xa_h, src_h, dst_h, et_h, zer_h, out_h,
                 esv, edv, etv, locb, rows, acc, gsem):
    c = lax.axis_index("c")
    s = lax.axis_index("s")
    dummy = R + s
    for p in range(2):
        r = 2 * p + c
        lo = r * R
        pltpu.sync_copy(zer_h, acc.at[pl.ds(s * ZSPAN, ZSPAN)])
        plsc.subcore_barrier()

        def sup(k2, carry):
            eb0 = s * EPP + k2 * EB
            pltpu.sync_copy(src_h.at[pl.ds(eb0, EB)], esv)
            pltpu.sync_copy(dst_h.at[pl.ds(eb0, EB)], edv)
            pltpu.sync_copy(et_h.at[pl.ds(eb0, EB)], etv)
            for j in range(EB // CHUNK):
                _compute_loc(edv, etv, locb, j, lo, dummy)
                pltpu.async_copy(xa_h.at[esv.at[pl.ds(j * CHUNK, CHUNK)]],
                                 rows, gsem).wait()
                pltpu.sync_copy(rows, acc.at[locb], add=True)
            return carry

        lax.fori_loop(0, EPP // EB, sup, 0)
        plsc.subcore_barrier()
        pltpu.sync_copy(acc.at[pl.ds(s * WSPAN, WSPAN)],
                        out_h.at[pl.ds(r * R + s * WSPAN, WSPAN)])

        @pl.when(s == 0)
        def _tail():
            pltpu.sync_copy(acc.at[pl.ds(NS * WSPAN, WTAIL)],
                            out_h.at[pl.ds(r * R + NS * WSPAN, WTAIL)])
        plsc.subcore_barrier()


def _sc_agg(xa, srcp, dstp, etp, zer):
    ka = pl.kernel(
        _sc_agg_body,
        out_type=_SDS((2 * N, D), jnp.float32),
        mesh=plsc.VectorSubcoreMesh(core_axis_name="c", subcore_axis_name="s"),
        scratch_types=(
            pltpu.VMEM((EB,), jnp.int32),
            pltpu.VMEM((EB,), jnp.int32),
            pltpu.VMEM((EB,), jnp.int32),
            pltpu.VMEM((CHUNK,), jnp.int32),
            pltpu.VMEM((CHUNK, D), jnp.float32),
            pltpu.VMEM_SHARED((ACCR, D), jnp.float32),
            pltpu.SemaphoreType.DMA,
        ),
    )
    return ka(xa, srcp, dstp, etp, zer)


def _sc_cnt_body(dst_h, et_h, ones_h, zer_h, out_h,
                 edv, etv, locb, onev, acc):
    c = lax.axis_index("c")
    s = lax.axis_index("s")
    dummy = R + s
    pltpu.sync_copy(ones_h, onev)
    for p in range(2):
        r = 2 * p + c
        lo = r * R
        pltpu.sync_copy(zer_h, acc.at[pl.ds(s * ZSPAN, ZSPAN)])
        plsc.subcore_barrier()

        def sup(k2, carry):
            eb0 = s * EPP + k2 * EB
            pltpu.sync_copy(dst_h.at[pl.ds(eb0, EB)], edv)
            pltpu.sync_copy(et_h.at[pl.ds(eb0, EB)], etv)
            for j in range(EB // CHUNK):
                _compute_loc(edv, etv, locb, j, lo, dummy)
                pltpu.sync_copy(onev, acc.at[locb], add=True)
            return carry

        lax.fori_loop(0, EPP // EB, sup, 0)
        plsc.subcore_barrier()
        pltpu.sync_copy(acc.at[pl.ds(s * WSPAN, WSPAN)],
                        out_h.at[pl.ds(r * R + s * WSPAN, WSPAN)])

        @pl.when(s == 0)
        def _tail():
            pltpu.sync_copy(acc.at[pl.ds(NS * WSPAN, WTAIL)],
                            out_h.at[pl.ds(r * R + NS * WSPAN, WTAIL)])
        plsc.subcore_barrier()


def _sc_cnt(dstp, etp, ones, zer):
    ka = pl.kernel(
        _sc_cnt_body,
        out_type=_SDS((2 * N, D), jnp.float32),
        mesh=plsc.VectorSubcoreMesh(core_axis_name="c", subcore_axis_name="s"),
        scratch_types=(
            pltpu.VMEM((EB,), jnp.int32),
            pltpu.VMEM((EB,), jnp.int32),
            pltpu.VMEM((CHUNK,), jnp.int32),
            pltpu.VMEM((CHUNK, D), jnp.float32),
            pltpu.VMEM_SHARED((ACCR, D), jnp.float32),
        ),
    )
    return ka(dstp, etp, ones, zer)


def kernel(des, tweet, num_prop, cat_prop, edge_index, edge_type,
           W_des, b_des, W_num, b_num, W_cat, b_cat, W_tweet, b_tweet,
           W_in, b_in, W_root, W_rel, b_rgcn,
           W_out1, b_out1, W_out2, b_out2):
    f32 = jnp.float32
    U = jnp.concatenate([des, num_prop, cat_prop], axis=1)
    U = jnp.pad(U, ((0, 0), (0, 128 - 117)))
    T = jnp.pad(tweet, ((0, 0), (0, 128 - 100)))
    P = jnp.concatenate([U, T], axis=0)
    Wbd = jnp.zeros((128, 128), f32)
    Wbd = Wbd.at[0:100, 0:64].set(W_des)
    Wbd = Wbd.at[100:106, 64:96].set(W_num)
    Wbd = Wbd.at[106:117, 96:128].set(W_cat)
    W1 = jnp.stack([Wbd, jnp.pad(W_tweet, ((0, 28), (0, 0)))])
    b1 = jnp.stack([jnp.concatenate([b_des, b_num, b_cat]),
                    b_tweet]).reshape(2, 1, 128)
    b_in2 = b_in.reshape(1, 128)
    b_r2 = b_rgcn.reshape(1, 128)
    bo1 = b_out1.reshape(1, 128)
    bo2 = b_out2.reshape(1, 2)
    Wr0 = W_rel[0]
    Wr1 = W_rel[1]
    npad = E_PAD - E
    srcp = jnp.pad(edge_index[0], (0, npad))
    dstp = jnp.pad(edge_index[1], (0, npad))
    etp = jnp.pad(edge_type, (0, npad), constant_values=2)
    zer = jnp.zeros((ZSPAN, D), f32)
    ones = jnp.zeros((CHUNK, D), f32).at[:, 0].set(1.0)

    xa0 = _prep(P, W1, b1, W_in, b_in2)
    cnt = _sc_cnt(dstp, etp, ones, zer)
    acc1 = _sc_agg(xa0, srcp, dstp, etp, zer)
    xa1 = _combine(False, xa0, acc1, cnt, W_root, Wr0, Wr1, b_r2,
                   W_out1, bo1, W_out2, bo2)
    acc2 = _sc_agg(xa1, srcp, dstp, etp, zer)
    y = _combine(True, xa1, acc2, cnt, W_root, Wr0, Wr1, b_r2,
                 W_out1, bo1, W_out2, bo2)
    return y
